# initial kernel scaffold (unmeasured)
import jax
import jax.numpy as jnp
from jax import lax
from jax.experimental import pallas as pl
from jax.experimental.pallas import tpu as pltpu

N_DEV = 32


def kernel(x, router_W, route_idx, expert_W, shared_W):
    n_tok, d_model = x.shape
    e_loc = expert_W.shape[0]
    d_ff = expert_W.shape[2]

    def body(x_ref, rw_ref, idx_ref, ew_ref, sw_ref, out_ref,
             comm_x, comm_meta, comm_acc,
             sx, rx, sm, rm, sa, ra, so, ro, credit):
        me = lax.axis_index("i")
        left = lax.rem(me - 1 + N_DEV, N_DEV)
        right = lax.rem(me + 1, N_DEV)

        barrier = pltpu.get_barrier_semaphore()
        for nbr in (left, right):
            pl.semaphore_signal(
                barrier, inc=1,
                device_id=(nbr,), device_id_type=pl.DeviceIdType.MESH,
            )
        pl.semaphore_wait(barrier, 2)

        xb0 = x_ref[:, :]
        scores = jnp.dot(xb0, rw_ref[:, :], preferred_element_type=jnp.float32)
        smax = jnp.max(scores, axis=-1, keepdims=True)
        ex = jnp.exp(scores - smax)
        probs = ex / jnp.sum(ex, axis=-1, keepdims=True)
        idx = idx_ref[:, 0]
        onehot = lax.broadcasted_iota(jnp.int32, scores.shape, 1) == idx[:, None]
        p = jnp.sum(jnp.where(onehot, probs, 0.0), axis=-1)

        comm_x[0, :, :] = xb0
        comm_meta[0, 0, :] = idx.astype(jnp.float32)
        comm_meta[0, 1, :] = p
        comm_acc[0, :, :] = jnp.dot(
            xb0, sw_ref[:, :], preferred_element_type=jnp.float32
        )

        for h in range(N_DEV):
            slot = h % 2
            nxt = (h + 1) % 2

            xb = comm_x[slot, :, :]
            bi = comm_meta[slot, 0, :]
            bp = comm_meta[slot, 1, :]
            acc = comm_acc[slot, :, :]
            for j in range(e_loc):
                full = jnp.dot(
                    xb, ew_ref[j, :, :], preferred_element_type=jnp.float32
                )
                eid = (me * e_loc + j).astype(jnp.float32)
                sel = jnp.where(bi == eid, bp, 0.0)
                acc = acc + sel[:, None] * full
            comm_acc[slot, :, :] = acc

            if h >= 1:
                pl.semaphore_wait(credit, 1)

            if h < N_DEV - 1:
                rdmas = [
                    pltpu.make_async_remote_copy(
                        src_ref=comm_x.at[slot], dst_ref=comm_x.at[nxt],
                        send_sem=sx.at[slot], recv_sem=rx.at[nxt],
                        device_id=(right,),
                        device_id_type=pl.DeviceIdType.MESH,
                    ),
                    pltpu.make_async_remote_copy(
                        src_ref=comm_meta.at[slot], dst_ref=comm_meta.at[nxt],
                        send_sem=sm.at[slot], recv_sem=rm.at[nxt],
                        device_id=(right,),
                        device_id_type=pl.DeviceIdType.MESH,
                    ),
                    pltpu.make_async_remote_copy(
                        src_ref=comm_acc.at[slot], dst_ref=comm_acc.at[nxt],
                        send_sem=sa.at[slot], recv_sem=ra.at[nxt],
                        device_id=(right,),
                        device_id_type=pl.DeviceIdType.MESH,
                    ),
                ]
                for r in rdmas:
                    r.start()
                for r in rdmas:
                    r.wait()
                pl.semaphore_signal(
                    credit, inc=1,
                    device_id=(left,), device_id_type=pl.DeviceIdType.MESH,
                )
            else:
                rf = pltpu.make_async_remote_copy(
                    src_ref=comm_acc.at[slot], dst_ref=out_ref,
                    send_sem=so, recv_sem=ro,
                    device_id=(right,),
                    device_id_type=pl.DeviceIdType.MESH,
                )
                rf.start()
                rf.wait()

    return pl.pallas_call(
        body,
        out_shape=jax.ShapeDtypeStruct((n_tok, d_ff), jnp.float32),
        in_specs=[
            pl.BlockSpec(memory_space=pltpu.VMEM),
            pl.BlockSpec(memory_space=pltpu.VMEM),
            pl.BlockSpec(memory_space=pltpu.VMEM),
            pl.BlockSpec(memory_space=pltpu.VMEM),
            pl.BlockSpec(memory_space=pltpu.VMEM),
        ],
        out_specs=pl.BlockSpec(memory_space=pltpu.VMEM),
        scratch_shapes=[
            pltpu.VMEM((2, n_tok, d_model), jnp.float32),
            pltpu.VMEM((2, 2, n_tok), jnp.float32),
            pltpu.VMEM((2, n_tok, d_ff), jnp.float32),
            pltpu.SemaphoreType.DMA((2,)),
            pltpu.SemaphoreType.DMA((2,)),
            pltpu.SemaphoreType.DMA((2,)),
            pltpu.SemaphoreType.DMA((2,)),
            pltpu.SemaphoreType.DMA((2,)),
            pltpu.SemaphoreType.DMA((2,)),
            pltpu.SemaphoreType.DMA,
            pltpu.SemaphoreType.DMA,
            pltpu.SemaphoreType.REGULAR,
        ],
        compiler_params=pltpu.CompilerParams(collective_id=0),
    )(x, router_W, route_idx, expert_W, shared_W)


# baseline (device time: 2383038 ns/iter reference)
import jax
import jax.numpy as jnp
from jax import lax
from jax.experimental import pallas as pl
from jax.experimental.pallas import tpu as pltpu

N_DEV = 32


def kernel(x, router_W, route_idx, expert_W, shared_W):
    n_tok, d_model = x.shape
    e_loc = expert_W.shape[0]
    d_ff = expert_W.shape[2]

    def body(x_ref, rw_ref, idx_ref, ew_ref, sw_ref, out_ref,
             comm_x, comm_meta, comm_acc,
             sx, rx, sm, rm, sa, ra, so, ro, credit):
        me = lax.axis_index("i")
        left = lax.rem(me - 1 + N_DEV, N_DEV)
        right = lax.rem(me + 1, N_DEV)

        barrier = pltpu.get_barrier_semaphore()
        for nbr in (left, right):
            pl.semaphore_signal(
                barrier, inc=1,
                device_id=(nbr,), device_id_type=pl.DeviceIdType.MESH,
            )
        pl.semaphore_wait(barrier, 2)

        xb0 = x_ref[:, :]
        scores = jnp.dot(xb0, rw_ref[:, :], preferred_element_type=jnp.float32)
        smax = jnp.max(scores, axis=-1, keepdims=True)
        ex = jnp.exp(scores - smax)
        probs = ex / jnp.sum(ex, axis=-1, keepdims=True)
        idx = idx_ref[:, 0]
        onehot = lax.broadcasted_iota(jnp.int32, scores.shape, 1) == idx[:, None]
        p = jnp.sum(jnp.where(onehot, probs, 0.0), axis=-1)

        comm_x[0, :, :] = xb0
        comm_meta[0, 0, :] = idx.astype(jnp.float32)
        comm_meta[0, 1, :] = p
        comm_acc[0, :, :] = jnp.dot(
            xb0, sw_ref[:, :], preferred_element_type=jnp.float32
        )

        def accumulate(slot):
            xb = comm_x[slot, :, :]
            bi = comm_meta[slot, 0, :]
            bp = comm_meta[slot, 1, :]
            acc = comm_acc[slot, :, :]
            for j in range(e_loc):
                full = jnp.dot(
                    xb, ew_ref[j, :, :], preferred_element_type=jnp.float32
                )
                eid = (me * e_loc + j).astype(jnp.float32)
                sel = jnp.where(bi == eid, bp, 0.0)
                acc = acc + sel[:, None] * full
            comm_acc[slot, :, :] = acc

        def forward(slot, nxt):
            rdmas = [
                pltpu.make_async_remote_copy(
                    src_ref=comm_x.at[slot], dst_ref=comm_x.at[nxt],
                    send_sem=sx.at[slot], recv_sem=rx.at[nxt],
                    device_id=(right,), device_id_type=pl.DeviceIdType.MESH,
                ),
                pltpu.make_async_remote_copy(
                    src_ref=comm_meta.at[slot], dst_ref=comm_meta.at[nxt],
                    send_sem=sm.at[slot], recv_sem=rm.at[nxt],
                    device_id=(right,), device_id_type=pl.DeviceIdType.MESH,
                ),
                pltpu.make_async_remote_copy(
                    src_ref=comm_acc.at[slot], dst_ref=comm_acc.at[nxt],
                    send_sem=sa.at[slot], recv_sem=ra.at[nxt],
                    device_id=(right,), device_id_type=pl.DeviceIdType.MESH,
                ),
            ]
            for r in rdmas:
                r.start()
            for r in rdmas:
                r.wait()
            pl.semaphore_signal(
                credit, inc=1,
                device_id=(left,), device_id_type=pl.DeviceIdType.MESH,
            )

        accumulate(0)
        forward(0, 1)

        def hop_body(h, carry):
            slot = lax.rem(h, 2)
            nxt = 1 - slot
            accumulate(slot)
            pl.semaphore_wait(credit, 1)
            forward(slot, nxt)
            return carry

        lax.fori_loop(1, N_DEV - 1, hop_body, 0)

        last = (N_DEV - 1) % 2
        accumulate(last)
        pl.semaphore_wait(credit, 1)
        rf = pltpu.make_async_remote_copy(
            src_ref=comm_acc.at[last], dst_ref=out_ref,
            send_sem=so, recv_sem=ro,
            device_id=(right,), device_id_type=pl.DeviceIdType.MESH,
        )
        rf.start()
        rf.wait()

    return pl.pallas_call(
        body,
        out_shape=jax.ShapeDtypeStruct((n_tok, d_ff), jnp.float32),
        in_specs=[
            pl.BlockSpec(memory_space=pltpu.VMEM),
            pl.BlockSpec(memory_space=pltpu.VMEM),
            pl.BlockSpec(memory_space=pltpu.VMEM),
            pl.BlockSpec(memory_space=pltpu.VMEM),
            pl.BlockSpec(memory_space=pltpu.VMEM),
        ],
        out_specs=pl.BlockSpec(memory_space=pltpu.VMEM),
        scratch_shapes=[
            pltpu.VMEM((2, n_tok, d_model), jnp.float32),
            pltpu.VMEM((2, 2, n_tok), jnp.float32),
            pltpu.VMEM((2, n_tok, d_ff), jnp.float32),
            pltpu.SemaphoreType.DMA((2,)),
            pltpu.SemaphoreType.DMA((2,)),
            pltpu.SemaphoreType.DMA((2,)),
            pltpu.SemaphoreType.DMA((2,)),
            pltpu.SemaphoreType.DMA((2,)),
            pltpu.SemaphoreType.DMA((2,)),
            pltpu.SemaphoreType.DMA,
            pltpu.SemaphoreType.DMA,
            pltpu.SemaphoreType.REGULAR,
        ],
        compiler_params=pltpu.CompilerParams(collective_id=0),
    )(x, router_W, route_idx, expert_W, shared_W)


# device time: 259667 ns/iter; 9.1773x vs baseline; 9.1773x over previous
import jax
import jax.numpy as jnp
from jax import lax
from jax.experimental import pallas as pl
from jax.experimental.pallas import tpu as pltpu

N_DEV = 32
CAP = 64


def kernel(x, router_W, route_idx, expert_W, shared_W):
    n_tok, d_model = x.shape
    e_loc = expert_W.shape[0]
    d_ff = expert_W.shape[2]
    n_slot = N_DEV * CAP

    scores = x @ router_W
    probs = jax.nn.softmax(scores, axis=-1)
    e = route_idx[:, 0]
    p = jnp.take_along_axis(probs, route_idx, axis=1)[:, 0]
    dest = e // e_loc
    jj = (e % e_loc).astype(jnp.float32)
    onehot = (dest[:, None] == jnp.arange(N_DEV)[None, :]).astype(jnp.int32)
    rank = jnp.take_along_axis(
        jnp.cumsum(onehot, axis=0) - 1, dest[:, None], axis=1
    )[:, 0]
    slot = jnp.where(rank < CAP, dest * CAP + rank, n_slot)
    slot_in = slot.astype(jnp.float32)[:, None]
    meta_in = jnp.concatenate(
        [jj[:, None], p[:, None], jnp.zeros((n_tok, 126), jnp.float32)], axis=1
    )

    def body(x_ref, slot_ref, meta_ref, ew_ref, sw_ref, out_ref,
             send_x, send_m, recv_x, recv_m, res, comb,
             dsx_s, dsx_r, dsm_s, dsm_r, cmb_s, cmb_r):
        me = lax.axis_index("i")
        slotv = slot_ref[:, 0].astype(jnp.int32)

        pt = (
            lax.broadcasted_iota(jnp.int32, (n_slot, n_tok), 0)
            == slotv[None, :]
        ).astype(jnp.float32)
        send_x[:, :] = jnp.dot(pt, x_ref[:, :],
                               preferred_element_type=jnp.float32)
        send_m[:, :] = jnp.dot(pt, meta_ref[:, :],
                               preferred_element_type=jnp.float32)

        sends = []
        for k in range(1, N_DEV):
            dd = lax.rem(me + k, N_DEV)
            for (src_buf, dst_buf, ssem, rsem) in (
                (send_x, recv_x, dsx_s, dsx_r),
                (send_m, recv_m, dsm_s, dsm_r),
            ):
                r = pltpu.make_async_remote_copy(
                    src_ref=src_buf.at[pl.ds(dd * CAP, CAP)],
                    dst_ref=dst_buf.at[pl.ds(me * CAP, CAP)],
                    send_sem=ssem.at[k],
                    recv_sem=rsem.at[me],
                    device_id=(dd,),
                    device_id_type=pl.DeviceIdType.MESH,
                )
                r.start()
                sends.append(r)
        recv_x[pl.ds(me * CAP, CAP), :] = send_x[pl.ds(me * CAP, CAP), :]
        recv_m[pl.ds(me * CAP, CAP), :] = send_m[pl.ds(me * CAP, CAP), :]

        shared = jnp.dot(x_ref[:, :], sw_ref[:, :],
                         preferred_element_type=jnp.float32)

        for k in range(1, N_DEV):
            src = lax.rem(me - k + N_DEV, N_DEV)
            for (dst_buf, rsem, ssem) in (
                (recv_x, dsx_r, dsx_s),
                (recv_m, dsm_r, dsm_s),
            ):
                pltpu.make_async_remote_copy(
                    src_ref=dst_buf.at[pl.ds(0, CAP)],
                    dst_ref=dst_buf.at[pl.ds(src * CAP, CAP)],
                    send_sem=ssem.at[k],
                    recv_sem=rsem.at[src],
                    device_id=(me,),
                    device_id_type=pl.DeviceIdType.MESH,
                ).wait_recv()

        rx = recv_x[:, :]
        jjv = recv_m[:, 0]
        ppv = recv_m[:, 1]
        acc = jnp.zeros((n_slot, d_ff), jnp.float32)
        for j in range(e_loc):
            full = jnp.dot(rx, ew_ref[j, :, :],
                           preferred_element_type=jnp.float32)
            sel = jnp.where(jjv == jnp.float32(j), ppv, 0.0)
            acc = acc + sel[:, None] * full
        res[:, :] = acc

        for k in range(1, N_DEV):
            dd = lax.rem(me + k, N_DEV)
            r = pltpu.make_async_remote_copy(
                src_ref=res.at[pl.ds(dd * CAP, CAP)],
                dst_ref=comb.at[pl.ds(me * CAP, CAP)],
                send_sem=cmb_s.at[k],
                recv_sem=cmb_r.at[me],
                device_id=(dd,),
                device_id_type=pl.DeviceIdType.MESH,
            )
            r.start()
            sends.append(r)
        comb[pl.ds(me * CAP, CAP), :] = res[pl.ds(me * CAP, CAP), :]

        for k in range(1, N_DEV):
            src = lax.rem(me - k + N_DEV, N_DEV)
            pltpu.make_async_remote_copy(
                src_ref=comb.at[pl.ds(0, CAP)],
                dst_ref=comb.at[pl.ds(src * CAP, CAP)],
                send_sem=cmb_s.at[k],
                recv_sem=cmb_r.at[src],
                device_id=(me,),
                device_id_type=pl.DeviceIdType.MESH,
            ).wait_recv()

        pmat = (
            lax.broadcasted_iota(jnp.int32, (n_tok, n_slot), 1)
            == slotv[:, None]
        ).astype(jnp.float32)
        out_ref[:, :] = shared + jnp.dot(
            pmat, comb[:, :], preferred_element_type=jnp.float32
        )

        for r in sends:
            r.wait_send()

    return pl.pallas_call(
        body,
        out_shape=jax.ShapeDtypeStruct((n_tok, d_ff), jnp.float32),
        in_specs=[pl.BlockSpec(memory_space=pltpu.VMEM)] * 5,
        out_specs=pl.BlockSpec(memory_space=pltpu.VMEM),
        scratch_shapes=[
            pltpu.VMEM((n_slot, d_model), jnp.float32),
            pltpu.VMEM((n_slot, 128), jnp.float32),
            pltpu.VMEM((n_slot, d_model), jnp.float32),
            pltpu.VMEM((n_slot, 128), jnp.float32),
            pltpu.VMEM((n_slot, d_ff), jnp.float32),
            pltpu.VMEM((n_slot, d_ff), jnp.float32),
            pltpu.SemaphoreType.DMA((N_DEV,)),
            pltpu.SemaphoreType.DMA((N_DEV,)),
            pltpu.SemaphoreType.DMA((N_DEV,)),
            pltpu.SemaphoreType.DMA((N_DEV,)),
            pltpu.SemaphoreType.DMA((N_DEV,)),
            pltpu.SemaphoreType.DMA((N_DEV,)),
        ],
        compiler_params=pltpu.CompilerParams(
            vmem_limit_bytes=100 * 1024 * 1024,
        ),
    )(x, slot_in, meta_in, expert_W, shared_W)


# device time: 165318 ns/iter; 14.4149x vs baseline; 1.5707x over previous
import jax
import jax.numpy as jnp
from jax import lax
from jax.experimental import pallas as pl
from jax.experimental.pallas import tpu as pltpu

N_DEV = 32
CAP = 64
XM = 640


def kernel(x, router_W, route_idx, expert_W, shared_W):
    n_tok, d_model = x.shape
    e_loc = expert_W.shape[0]
    d_ff = expert_W.shape[2]
    n_slot = N_DEV * CAP

    scores = x @ router_W
    probs = jax.nn.softmax(scores, axis=-1)
    e = route_idx[:, 0]
    p = jnp.take_along_axis(probs, route_idx, axis=1)[:, 0]
    dest = e // e_loc
    jj = (e % e_loc).astype(jnp.float32)
    onehot = (dest[:, None] == jnp.arange(N_DEV)[None, :]).astype(jnp.int32)
    rank = jnp.take_along_axis(
        jnp.cumsum(onehot, axis=0) - 1, dest[:, None], axis=1
    )[:, 0]
    slot = jnp.where(rank < CAP, dest * CAP + rank, n_slot)
    slot_in = slot.astype(jnp.float32)[:, None]
    xm = jnp.concatenate(
        [x, jj[:, None], p[:, None],
         jnp.zeros((n_tok, XM - d_model - 2), jnp.float32)], axis=1
    )

    def body(xm_ref, slot_ref, ew_ref, sw_ref, out_ref,
             send_b, recv_b, ew_cat, res, comb,
             dsp_s, dsp_r, cmb_s, cmb_r):
        me = lax.axis_index("i")
        slotv = slot_ref[:, 0].astype(jnp.int32)

        pt = (
            lax.broadcasted_iota(jnp.int32, (n_slot, n_tok), 0)
            == slotv[None, :]
        ).astype(jnp.float32)
        send_b[:, :] = jnp.dot(
            pt, xm_ref[:, :], preferred_element_type=jnp.float32
        ).astype(jnp.bfloat16)

        for k in range(1, N_DEV):
            dd = lax.rem(me + k, N_DEV)
            pltpu.make_async_remote_copy(
                src_ref=send_b.at[pl.ds(dd * CAP, CAP)],
                dst_ref=recv_b.at[pl.ds(me * CAP, CAP)],
                send_sem=dsp_s.at[k],
                recv_sem=dsp_r.at[me],
                device_id=(dd,),
                device_id_type=pl.DeviceIdType.MESH,
            ).start()
        recv_b[pl.ds(me * CAP, CAP), :] = send_b[pl.ds(me * CAP, CAP), :]

        for j in range(e_loc):
            ew_cat[:, pl.ds(j * d_ff, d_ff)] = (
                ew_ref[j, :, :].astype(jnp.bfloat16)
            )

        shared = jnp.dot(
            xm_ref[:, :d_model], sw_ref[:, :],
            preferred_element_type=jnp.float32,
        )

        def chunk_compute(src):
            rows = pl.ds(src * CAP, CAP)
            rxc = recv_b[rows, :d_model]
            jv = recv_b[rows, d_model]
            pv = recv_b[rows, d_model + 1]
            full = jnp.dot(
                rxc, ew_cat[:, :], preferred_element_type=jnp.float32
            )
            acc = jnp.zeros((CAP, d_ff), jnp.float32)
            for j in range(e_loc):
                sel = jnp.where(jv == jnp.bfloat16(j), pv, jnp.bfloat16(0))
                acc = acc + sel.astype(jnp.float32)[:, None] * full[
                    :, j * d_ff:(j + 1) * d_ff
                ]
            res[rows, :] = acc.astype(jnp.bfloat16)

        chunk_compute(me)
        comb[pl.ds(me * CAP, CAP), :] = res[pl.ds(me * CAP, CAP), :]

        def pipe_body(k, carry):
            src = lax.rem(me - k + N_DEV, N_DEV)
            pltpu.make_async_remote_copy(
                src_ref=send_b.at[pl.ds(0, CAP)],
                dst_ref=recv_b.at[pl.ds(src * CAP, CAP)],
                send_sem=dsp_s.at[0],
                recv_sem=dsp_r.at[src],
                device_id=(me,),
                device_id_type=pl.DeviceIdType.MESH,
            ).wait_recv()
            chunk_compute(src)
            pltpu.make_async_remote_copy(
                src_ref=res.at[pl.ds(src * CAP, CAP)],
                dst_ref=comb.at[pl.ds(me * CAP, CAP)],
                send_sem=cmb_s.at[src],
                recv_sem=cmb_r.at[me],
                device_id=(src,),
                device_id_type=pl.DeviceIdType.MESH,
            ).start()
            return carry

        lax.fori_loop(1, N_DEV, pipe_body, 0)

        def cwait_body(k, carry):
            src = lax.rem(me - k + N_DEV, N_DEV)
            pltpu.make_async_remote_copy(
                src_ref=res.at[pl.ds(0, CAP)],
                dst_ref=comb.at[pl.ds(src * CAP, CAP)],
                send_sem=cmb_s.at[me],
                recv_sem=cmb_r.at[src],
                device_id=(me,),
                device_id_type=pl.DeviceIdType.MESH,
            ).wait_recv()
            return carry

        lax.fori_loop(1, N_DEV, cwait_body, 0)

        pmat = (
            lax.broadcasted_iota(jnp.int32, (n_tok, n_slot), 1)
            == slotv[:, None]
        ).astype(jnp.bfloat16)
        out_ref[:, :] = shared + jnp.dot(
            pmat, comb[:, :], preferred_element_type=jnp.float32
        )

        for k in range(1, N_DEV):
            pltpu.make_async_remote_copy(
                src_ref=send_b.at[pl.ds(0, CAP)],
                dst_ref=recv_b.at[pl.ds(0, CAP)],
                send_sem=dsp_s.at[k],
                recv_sem=dsp_r.at[me],
                device_id=(me,),
                device_id_type=pl.DeviceIdType.MESH,
            ).wait_send()

        def dwait_body(k, carry):
            src = lax.rem(me - k + N_DEV, N_DEV)
            pltpu.make_async_remote_copy(
                src_ref=res.at[pl.ds(0, CAP)],
                dst_ref=comb.at[pl.ds(0, CAP)],
                send_sem=cmb_s.at[src],
                recv_sem=cmb_r.at[me],
                device_id=(me,),
                device_id_type=pl.DeviceIdType.MESH,
            ).wait_send()
            return carry

        lax.fori_loop(1, N_DEV, dwait_body, 0)

    return pl.pallas_call(
        body,
        out_shape=jax.ShapeDtypeStruct((n_tok, d_ff), jnp.float32),
        in_specs=[pl.BlockSpec(memory_space=pltpu.VMEM)] * 4,
        out_specs=pl.BlockSpec(memory_space=pltpu.VMEM),
        scratch_shapes=[
            pltpu.VMEM((n_slot, XM), jnp.bfloat16),
            pltpu.VMEM((n_slot, XM), jnp.bfloat16),
            pltpu.VMEM((d_model, e_loc * d_ff), jnp.bfloat16),
            pltpu.VMEM((n_slot, d_ff), jnp.bfloat16),
            pltpu.VMEM((n_slot, d_ff), jnp.bfloat16),
            pltpu.SemaphoreType.DMA((N_DEV,)),
            pltpu.SemaphoreType.DMA((N_DEV,)),
            pltpu.SemaphoreType.DMA((N_DEV,)),
            pltpu.SemaphoreType.DMA((N_DEV,)),
        ],
        compiler_params=pltpu.CompilerParams(
            vmem_limit_bytes=100 * 1024 * 1024,
        ),
    )(xm, slot_in, expert_W, shared_W)
